# baseline (speedup 1.0000x reference)
import functools

import jax
import jax.numpy as jnp
from jax import lax
from jax.experimental import pallas as pl
from jax.experimental.pallas import tpu as pltpu
from jax.experimental.pallas import tpu_sc as plsc

_N = 16384
_R = 3
_NC = 2
_NS = 16
_NW = _NC * _NS
_B = _N // _NW
_E = _B * _R
_L = 16

_mesh = plsc.VectorSubcoreMesh(core_axis_name="c", subcore_axis_name="s")


@functools.partial(
    pl.kernel,
    mesh=_mesh,
    compiler_params=pltpu.CompilerParams(needs_layout_passes=False),
    out_type=jax.ShapeDtypeStruct((_N * _R,), jnp.float32),
    scratch_types=[
        pltpu.VMEM((_B,), jnp.int32),
        pltpu.VMEM((_E,), jnp.int32),
        pltpu.VMEM((_E,), jnp.float32),
        pltpu.SemaphoreType.DMA,
    ],
)
def _gather_rows(idx_hbm, table_hbm, out_hbm, idx_v, ind_v, rows_v, sem):
    wid = lax.axis_index("s") * _NC + lax.axis_index("c")
    base = wid * _B
    pltpu.sync_copy(idx_hbm.at[pl.ds(base, _B)], idx_v)
    lane3 = lax.iota(jnp.int32, _L) * 3
    for j in range(_B // _L):
        v3 = idx_v[pl.ds(j * _L, _L)] * 3
        pos = lane3 + (3 * _L * j)
        for r in range(_R):
            plsc.store_scatter(ind_v, [pos + r], v3 + r)
    pltpu.async_copy(table_hbm.at[ind_v], rows_v, sem).wait()
    pltpu.sync_copy(rows_v, out_hbm.at[pl.ds(base * _R, _E)])


def kernel(gene_index_tensor_n, cell_index_tensor_n, downsampling_rate_tensor_n,
           total_obs_reads_per_cell_tensor_n, cell_features_nf,
           global_prior_params_gr):
    idx = gene_index_tensor_n.astype(jnp.int32)
    table_flat = global_prior_params_gr.reshape(-1)
    out_flat = _gather_rows(idx, table_flat)
    return out_flat.reshape(_N, _R)


# speedup vs baseline: 3.7938x; 3.7938x over previous
import functools

import jax
import jax.numpy as jnp
from jax import lax
from jax.experimental import pallas as pl
from jax.experimental.pallas import tpu as pltpu
from jax.experimental.pallas import tpu_sc as plsc

_N = 16384
_G = 100000
_R = 3
_NC = 2
_NS = 16
_NW = _NC * _NS
_B = _N // _NW

_mesh = plsc.VectorSubcoreMesh(core_axis_name="c", subcore_axis_name="s")


@functools.partial(
    pl.kernel,
    mesh=_mesh,
    compiler_params=pltpu.CompilerParams(
        needs_layout_passes=False, use_tc_tiling_on_sc=False
    ),
    out_type=jax.ShapeDtypeStruct((_R, _N), jnp.float32),
    scratch_types=[
        pltpu.VMEM((_B,), jnp.int32),
        pltpu.VMEM((_B,), jnp.float32),
        pltpu.VMEM((_B,), jnp.float32),
        pltpu.VMEM((_B,), jnp.float32),
        pltpu.SemaphoreType.DMA,
    ],
)
def _gather_cols(idx_hbm, table_hbm, out_hbm, idx_v, r0, r1, r2, sem):
    wid = lax.axis_index("s") * _NC + lax.axis_index("c")
    base = wid * _B
    pltpu.sync_copy(idx_hbm.at[pl.ds(base, _B)], idx_v)
    pltpu.async_copy(table_hbm.at[0].at[idx_v], r0, sem).wait()
    pltpu.async_copy(table_hbm.at[1].at[idx_v], r1, sem).wait()
    pltpu.async_copy(table_hbm.at[2].at[idx_v], r2, sem).wait()
    pltpu.sync_copy(r0, out_hbm.at[0, pl.ds(base, _B)])
    pltpu.sync_copy(r1, out_hbm.at[1, pl.ds(base, _B)])
    pltpu.sync_copy(r2, out_hbm.at[2, pl.ds(base, _B)])


def kernel(gene_index_tensor_n, cell_index_tensor_n, downsampling_rate_tensor_n,
           total_obs_reads_per_cell_tensor_n, cell_features_nf,
           global_prior_params_gr):
    table_t = global_prior_params_gr.T
    out_t = _gather_cols(gene_index_tensor_n, table_t)
    return out_t.T


# speedup vs baseline: 4.0405x; 1.0650x over previous
import functools

import jax
import jax.numpy as jnp
from jax import lax
from jax.experimental import pallas as pl
from jax.experimental.pallas import tpu as pltpu
from jax.experimental.pallas import tpu_sc as plsc

_N = 16384
_G = 100000
_R = 3
_NC = 2
_NS = 16
_NW = _NC * _NS
_B = _N // _NW

_mesh = plsc.VectorSubcoreMesh(core_axis_name="c", subcore_axis_name="s")


@functools.partial(
    pl.kernel,
    mesh=_mesh,
    compiler_params=pltpu.CompilerParams(
        needs_layout_passes=False, use_tc_tiling_on_sc=False
    ),
    out_type=jax.ShapeDtypeStruct((_R, _N), jnp.float32),
    scratch_types=[
        pltpu.VMEM((_B,), jnp.int32),
        pltpu.VMEM((_B,), jnp.float32),
        pltpu.VMEM((_B,), jnp.float32),
        pltpu.VMEM((_B,), jnp.float32),
        pltpu.SemaphoreType.DMA,
        pltpu.SemaphoreType.DMA,
    ],
)
def _gather_cols(idx_hbm, table_hbm, out_hbm, idx_v, r0, r1, r2, sem, osem):
    wid = lax.axis_index("s") * _NC + lax.axis_index("c")
    base = wid * _B
    pltpu.sync_copy(idx_hbm.at[pl.ds(base, _B)], idx_v)
    g0 = pltpu.async_copy(table_hbm.at[0].at[idx_v], r0, sem)
    g1 = pltpu.async_copy(table_hbm.at[1].at[idx_v], r1, sem)
    g2 = pltpu.async_copy(table_hbm.at[2].at[idx_v], r2, sem)
    g0.wait()
    s0 = pltpu.async_copy(r0, out_hbm.at[0, pl.ds(base, _B)], osem)
    g1.wait()
    s1 = pltpu.async_copy(r1, out_hbm.at[1, pl.ds(base, _B)], osem)
    g2.wait()
    s2 = pltpu.async_copy(r2, out_hbm.at[2, pl.ds(base, _B)], osem)
    s0.wait()
    s1.wait()
    s2.wait()


def kernel(gene_index_tensor_n, cell_index_tensor_n, downsampling_rate_tensor_n,
           total_obs_reads_per_cell_tensor_n, cell_features_nf,
           global_prior_params_gr):
    table_t = global_prior_params_gr.T
    out_t = _gather_cols(gene_index_tensor_n, table_t)
    return out_t.T
